# baseline (speedup 1.0000x reference)
import functools

import jax
import jax.numpy as jnp
from jax import lax
from jax.experimental import pallas as pl
from jax.experimental.pallas import tpu as pltpu
from jax.experimental.pallas import tpu_sc as plsc

N = 100000
E = 6400000
F = 4
H = 32

NC = 2
NS = 16
NW = NC * NS
EW = E // NW
SUB = 128
K = 16
CH = SUB * K
PAD = 704
EWP = EW + PAD
NCHUNK = EWP // CH
E2 = NW * EWP
RB = N // 50


def _sc_body(src_hbm, dst_hbm, w_hbm, ce_hbm, x_hbm, zb_hbm,
             acc_out, cnt_out,
             a0_sp, a1_sp, a2_sp, a3_sp, cnt_sp,
             src_v, dst_v, w_v, cnt_v, rows_v, p0_v, p1_v, p2_v, p3_v,
             sem, sem2, sem3):
    pays = (p0_v, p1_v, p2_v, p3_v)
    accs = (a0_sp, a1_sp, a2_sp, a3_sp)
    cid = lax.axis_index("c")
    sid = lax.axis_index("s")
    wid = cid * NS + sid

    for k in range(4):
        b = sid + NS * k
        def _zero(b=b):
            for a in accs:
                pltpu.sync_copy(zb_hbm, a.at[pl.ds(b * RB, RB)])
            pltpu.sync_copy(zb_hbm, cnt_sp.at[pl.ds(b * RB, RB)])
        if k < 3:
            _zero()
        else:
            pl.when(sid < 2)(_zero)
    plsc.subcore_barrier()

    iota = lax.iota(jnp.int32, 16)

    def chunk_body(c, carry):
        eb = wid * EWP + c * CH
        lh = [pltpu.async_copy(src_hbm.at[pl.ds(eb, CH)], src_v, sem3),
              pltpu.async_copy(dst_hbm.at[pl.ds(eb, CH)], dst_v, sem3),
              pltpu.async_copy(w_hbm.at[pl.ds(eb, CH)], w_v, sem3),
              pltpu.async_copy(ce_hbm.at[pl.ds(eb, CH)], cnt_v, sem3)]
        for h in lh:
            h.wait()
        pltpu.async_copy(x_hbm.at[src_v], rows_v, sem).wait()

        def mul_body(i, carry2):
            base = i * 16
            evec = iota + base
            wv = w_v[pl.ds(base, 16)]
            for col in range(F):
                cc = jnp.full((16,), col, jnp.int32)
                xc = plsc.load_gather(rows_v, [evec, cc])
                pays[col][pl.ds(base, 16)] = xc * wv
            return carry2
        lax.fori_loop(0, CH // 16, mul_body, 0)

        for col in range(F):
            pltpu.sync_copy(pays[col], accs[col].at[dst_v], add=True)
        pltpu.sync_copy(cnt_v, cnt_sp.at[dst_v], add=True)
        return carry
    lax.fori_loop(0, NCHUNK, chunk_body, 0)
    plsc.subcore_barrier()

    for k in range(4):
        b = sid + NS * k
        def _drain(b=b):
            for col in range(F):
                pltpu.sync_copy(accs[col].at[pl.ds(b * RB, RB)],
                                acc_out.at[cid, col, pl.ds(b * RB, RB)])
            pltpu.sync_copy(cnt_sp.at[pl.ds(b * RB, RB)],
                            cnt_out.at[cid, pl.ds(b * RB, RB)])
        if k < 3:
            _drain()
        else:
            pl.when(sid < 2)(_drain)


def _sc_aggregate(src2d, dst2d, w, ce, x):
    zb = jnp.zeros((RB,), jnp.float32)
    mesh = plsc.VectorSubcoreMesh(core_axis_name="c", subcore_axis_name="s",
                                  num_cores=NC, num_subcores=NS)
    fn = pl.kernel(
        _sc_body,
        out_type=(jax.ShapeDtypeStruct((NC, F, N), jnp.float32),
                  jax.ShapeDtypeStruct((NC, N), jnp.float32)),
        mesh=mesh,
        scratch_types=(
            pltpu.VMEM_SHARED((N,), jnp.float32),
            pltpu.VMEM_SHARED((N,), jnp.float32),
            pltpu.VMEM_SHARED((N,), jnp.float32),
            pltpu.VMEM_SHARED((N,), jnp.float32),
            pltpu.VMEM_SHARED((N,), jnp.float32),
            pltpu.VMEM((CH,), jnp.int32),
            pltpu.VMEM((CH,), jnp.int32),
            pltpu.VMEM((CH,), jnp.float32),
            pltpu.VMEM((CH,), jnp.float32),
            pltpu.VMEM((CH, F), jnp.float32),
            pltpu.VMEM((CH,), jnp.float32),
            pltpu.VMEM((CH,), jnp.float32),
            pltpu.VMEM((CH,), jnp.float32),
            pltpu.VMEM((CH,), jnp.float32),
            pltpu.SemaphoreType.DMA,
            pltpu.SemaphoreType.DMA,
            pltpu.SemaphoreType.DMA,
        ),
        compiler_params=pltpu.CompilerParams(use_tc_tiling_on_sc=False,
                                             needs_layout_passes=False),
    )
    return fn(src2d, dst2d, w, ce, x, zb)


def _tc_body(acc_ref, cnt_ref, x_ref, ggc_ref, gih_ref, ghh_ref, gbi_ref,
             gbh_ref, lih_ref, lb_ref, linw_ref, linb_ref, out_ref):
    f32 = jnp.float32
    a = acc_ref[0] + acc_ref[1]
    cnt = cnt_ref[0] + cnt_ref[1]
    aggp = a / jnp.maximum(cnt, 1.0)
    agg = jnp.dot(aggp, ggc_ref[...], preferred_element_type=f32)
    x = x_ref[...]
    gi = jnp.dot(agg, gih_ref[...], preferred_element_type=f32) + gbi_ref[...]
    gh = jnp.dot(x, ghh_ref[...], preferred_element_type=f32) + gbh_ref[...]
    r = jax.nn.sigmoid(gi[:, 0:F] + gh[:, 0:F])
    z = jax.nn.sigmoid(gi[:, F:2 * F] + gh[:, F:2 * F])
    n = jnp.tanh(gi[:, 2 * F:3 * F] + r * gh[:, 2 * F:3 * F])
    h_tilde = (1.0 - z) * n + z * x
    gates = jnp.dot(h_tilde, lih_ref[...], preferred_element_type=f32) \
        + lb_ref[...]
    i_g = jax.nn.sigmoid(gates[:, 0:H])
    g_g = jnp.tanh(gates[:, 2 * H:3 * H])
    o_g = jax.nn.sigmoid(gates[:, 3 * H:4 * H])
    c = i_g * g_g
    h_out = o_g * jnp.tanh(c)
    out_ref[...] = jnp.dot(jnp.maximum(h_out, 0.0), linw_ref[...],
                           preferred_element_type=f32) + linb_ref[...]


def _tc_stage(acc, cnt3, x, ggc, gihT, ghhT, gbi, gbh, lihT, lb, linwT, linb):
    grid = (N // RB,)
    full = lambda shape: pl.BlockSpec(shape, lambda i: (0,) * len(shape))
    return pl.pallas_call(
        _tc_body,
        grid=grid,
        in_specs=[
            pl.BlockSpec((NC, RB, F), lambda i: (0, i, 0)),
            pl.BlockSpec((NC, RB, 1), lambda i: (0, i, 0)),
            pl.BlockSpec((RB, F), lambda i: (i, 0)),
            full((F, F)),
            full((F, 3 * F)),
            full((F, 3 * F)),
            full((1, 3 * F)),
            full((1, 3 * F)),
            full((F, 4 * H)),
            full((1, 4 * H)),
            full((H, 1)),
            full((1, 1)),
        ],
        out_specs=pl.BlockSpec((RB, 1), lambda i: (i, 0)),
        out_shape=jax.ShapeDtypeStruct((N, 1), jnp.float32),
    )(acc, cnt3, x, ggc, gihT, ghhT, gbi, gbh, lihT, lb, linwT, linb)


@jax.jit
def kernel(x, edge_index, edge_weight, ggc_weight, gru_w_ih, gru_w_hh,
           gru_b_ih, gru_b_hh, lstm_w_ih, lstm_w_hh, lstm_b_ih, lstm_b_hh,
           lin_w, lin_b):
    padw = ((0, 0), (0, PAD))
    src2d = jnp.pad(edge_index[0].reshape(NW, EW), padw).reshape(E2)
    dst2d = jnp.pad(edge_index[1].reshape(NW, EW), padw).reshape(E2)
    w_p = jnp.pad(edge_weight.reshape(NW, EW), padw).reshape(E2)
    ce_p = jnp.pad(jnp.ones((NW, EW), jnp.float32), padw).reshape(E2)
    accT, cnt = _sc_aggregate(src2d, dst2d, w_p, ce_p, x)
    acc = jnp.transpose(accT, (0, 2, 1))
    cnt3 = cnt[..., None]
    return _tc_stage(
        acc, cnt3, x,
        ggc_weight,
        gru_w_ih.T, gru_w_hh.T,
        gru_b_ih[None, :], gru_b_hh[None, :],
        lstm_w_ih.T, (lstm_b_ih + lstm_b_hh)[None, :],
        lin_w.T, lin_b[None, :],
    )


# speedup vs baseline: 1.0283x; 1.0283x over previous
import functools

import jax
import jax.numpy as jnp
from jax import lax
from jax.experimental import pallas as pl
from jax.experimental.pallas import tpu as pltpu
from jax.experimental.pallas import tpu_sc as plsc

N = 100000
E = 6400000
F = 4
H = 32

NC = 2
NS = 16
NW = NC * NS
EW = E // NW
SUB = 128
K = 32
CH = SUB * K
PAD = 704
EWP = EW + PAD
NCHUNK = EWP // CH
E2 = NW * EWP
RB = N // 50


def _sc_body(src_hbm, dst_hbm, w_hbm, ce_hbm, x_hbm, zb_hbm,
             acc_out, cnt_out,
             a0_sp, a1_sp, a2_sp, a3_sp, cnt_sp,
             src_v, dst_v, w_v, cnt_v, rows_v, p0_v, p1_v, p2_v, p3_v,
             sem, sem2, sem3):
    pays = (p0_v, p1_v, p2_v, p3_v)
    accs = (a0_sp, a1_sp, a2_sp, a3_sp)
    cid = lax.axis_index("c")
    sid = lax.axis_index("s")
    wid = cid * NS + sid

    for k in range(4):
        b = sid + NS * k
        def _zero(b=b):
            for a in accs:
                pltpu.sync_copy(zb_hbm, a.at[pl.ds(b * RB, RB)])
            pltpu.sync_copy(zb_hbm, cnt_sp.at[pl.ds(b * RB, RB)])
        if k < 3:
            _zero()
        else:
            pl.when(sid < 2)(_zero)
    plsc.subcore_barrier()

    iota = lax.iota(jnp.int32, 16)

    def chunk_body(c, carry):
        eb = wid * EWP + c * CH
        lh = [pltpu.async_copy(src_hbm.at[pl.ds(eb, CH)], src_v, sem3),
              pltpu.async_copy(dst_hbm.at[pl.ds(eb, CH)], dst_v, sem3),
              pltpu.async_copy(w_hbm.at[pl.ds(eb, CH)], w_v, sem3),
              pltpu.async_copy(ce_hbm.at[pl.ds(eb, CH)], cnt_v, sem3)]
        for h in lh:
            h.wait()
        pltpu.async_copy(x_hbm.at[src_v], rows_v, sem).wait()

        def mul_body(i, carry2):
            base = i * 16
            evec = iota + base
            wv = w_v[pl.ds(base, 16)]
            for col in range(F):
                cc = jnp.full((16,), col, jnp.int32)
                xc = plsc.load_gather(rows_v, [evec, cc])
                pays[col][pl.ds(base, 16)] = xc * wv
            return carry2
        lax.fori_loop(0, CH // 16, mul_body, 0)

        for col in range(F):
            pltpu.sync_copy(pays[col], accs[col].at[dst_v], add=True)
        pltpu.sync_copy(cnt_v, cnt_sp.at[dst_v], add=True)
        return carry
    lax.fori_loop(0, NCHUNK, chunk_body, 0)
    plsc.subcore_barrier()

    for k in range(4):
        b = sid + NS * k
        def _drain(b=b):
            for col in range(F):
                pltpu.sync_copy(accs[col].at[pl.ds(b * RB, RB)],
                                acc_out.at[cid, col, pl.ds(b * RB, RB)])
            pltpu.sync_copy(cnt_sp.at[pl.ds(b * RB, RB)],
                            cnt_out.at[cid, pl.ds(b * RB, RB)])
        if k < 3:
            _drain()
        else:
            pl.when(sid < 2)(_drain)


def _sc_aggregate(src2d, dst2d, w, ce, x):
    zb = jnp.zeros((RB,), jnp.float32)
    mesh = plsc.VectorSubcoreMesh(core_axis_name="c", subcore_axis_name="s",
                                  num_cores=NC, num_subcores=NS)
    fn = pl.kernel(
        _sc_body,
        out_type=(jax.ShapeDtypeStruct((NC, F, N), jnp.float32),
                  jax.ShapeDtypeStruct((NC, N), jnp.float32)),
        mesh=mesh,
        scratch_types=(
            pltpu.VMEM_SHARED((N,), jnp.float32),
            pltpu.VMEM_SHARED((N,), jnp.float32),
            pltpu.VMEM_SHARED((N,), jnp.float32),
            pltpu.VMEM_SHARED((N,), jnp.float32),
            pltpu.VMEM_SHARED((N,), jnp.float32),
            pltpu.VMEM((CH,), jnp.int32),
            pltpu.VMEM((CH,), jnp.int32),
            pltpu.VMEM((CH,), jnp.float32),
            pltpu.VMEM((CH,), jnp.float32),
            pltpu.VMEM((CH, F), jnp.float32),
            pltpu.VMEM((CH,), jnp.float32),
            pltpu.VMEM((CH,), jnp.float32),
            pltpu.VMEM((CH,), jnp.float32),
            pltpu.VMEM((CH,), jnp.float32),
            pltpu.SemaphoreType.DMA,
            pltpu.SemaphoreType.DMA,
            pltpu.SemaphoreType.DMA,
        ),
        compiler_params=pltpu.CompilerParams(use_tc_tiling_on_sc=False,
                                             needs_layout_passes=False),
    )
    return fn(src2d, dst2d, w, ce, x, zb)


def _tc_body(acc_ref, cnt_ref, x_ref, ggc_ref, gih_ref, ghh_ref, gbi_ref,
             gbh_ref, lih_ref, lb_ref, linw_ref, linb_ref, out_ref):
    f32 = jnp.float32
    a = acc_ref[0] + acc_ref[1]
    cnt = cnt_ref[0] + cnt_ref[1]
    aggp = a / jnp.maximum(cnt, 1.0)
    agg = jnp.dot(aggp, ggc_ref[...], preferred_element_type=f32)
    x = x_ref[...]
    gi = jnp.dot(agg, gih_ref[...], preferred_element_type=f32) + gbi_ref[...]
    gh = jnp.dot(x, ghh_ref[...], preferred_element_type=f32) + gbh_ref[...]
    r = jax.nn.sigmoid(gi[:, 0:F] + gh[:, 0:F])
    z = jax.nn.sigmoid(gi[:, F:2 * F] + gh[:, F:2 * F])
    n = jnp.tanh(gi[:, 2 * F:3 * F] + r * gh[:, 2 * F:3 * F])
    h_tilde = (1.0 - z) * n + z * x
    gates = jnp.dot(h_tilde, lih_ref[...], preferred_element_type=f32) \
        + lb_ref[...]
    i_g = jax.nn.sigmoid(gates[:, 0:H])
    g_g = jnp.tanh(gates[:, 2 * H:3 * H])
    o_g = jax.nn.sigmoid(gates[:, 3 * H:4 * H])
    c = i_g * g_g
    h_out = o_g * jnp.tanh(c)
    out_ref[...] = jnp.dot(jnp.maximum(h_out, 0.0), linw_ref[...],
                           preferred_element_type=f32) + linb_ref[...]


def _tc_stage(acc, cnt3, x, ggc, gihT, ghhT, gbi, gbh, lihT, lb, linwT, linb):
    grid = (N // RB,)
    full = lambda shape: pl.BlockSpec(shape, lambda i: (0,) * len(shape))
    return pl.pallas_call(
        _tc_body,
        grid=grid,
        in_specs=[
            pl.BlockSpec((NC, RB, F), lambda i: (0, i, 0)),
            pl.BlockSpec((NC, RB, 1), lambda i: (0, i, 0)),
            pl.BlockSpec((RB, F), lambda i: (i, 0)),
            full((F, F)),
            full((F, 3 * F)),
            full((F, 3 * F)),
            full((1, 3 * F)),
            full((1, 3 * F)),
            full((F, 4 * H)),
            full((1, 4 * H)),
            full((H, 1)),
            full((1, 1)),
        ],
        out_specs=pl.BlockSpec((RB, 1), lambda i: (i, 0)),
        out_shape=jax.ShapeDtypeStruct((N, 1), jnp.float32),
    )(acc, cnt3, x, ggc, gihT, ghhT, gbi, gbh, lihT, lb, linwT, linb)


@jax.jit
def kernel(x, edge_index, edge_weight, ggc_weight, gru_w_ih, gru_w_hh,
           gru_b_ih, gru_b_hh, lstm_w_ih, lstm_w_hh, lstm_b_ih, lstm_b_hh,
           lin_w, lin_b):
    padw = ((0, 0), (0, PAD))
    src2d = jnp.pad(edge_index[0].reshape(NW, EW), padw).reshape(E2)
    dst2d = jnp.pad(edge_index[1].reshape(NW, EW), padw).reshape(E2)
    w_p = jnp.pad(edge_weight.reshape(NW, EW), padw).reshape(E2)
    ce_p = jnp.pad(jnp.ones((NW, EW), jnp.float32), padw).reshape(E2)
    accT, cnt = _sc_aggregate(src2d, dst2d, w_p, ce_p, x)
    acc = jnp.transpose(accT, (0, 2, 1))
    cnt3 = cnt[..., None]
    return _tc_stage(
        acc, cnt3, x,
        ggc_weight,
        gru_w_ih.T, gru_w_hh.T,
        gru_b_ih[None, :], gru_b_hh[None, :],
        lstm_w_ih.T, (lstm_b_ih + lstm_b_hh)[None, :],
        lin_w.T, lin_b[None, :],
    )
